# baseline (speedup 1.0000x reference)
import functools

import jax
import jax.numpy as jnp
from jax import lax
from jax.experimental import pallas as pl
from jax.experimental.pallas import tpu as pltpu
from jax.experimental.pallas import tpu_sc as plsc

EMB_DIM = 64
LTM_N = 1000000
STM_CAP = 128
K = 3
RADIUS2 = 9.0
SIM_THRESHOLD = 0.7
EPS = 1e-8
NEG_INF = float("-inf")
BIG_I32 = 1 << 30
DN_T = (((1,), (1,)), ((), ()))

NSTREAM = 4
STEP_ROWS = 32768
SUB_ROWS = STEP_ROWS // NSTREAM
XSUB = SUB_ROWS // 2
TC_NBLK = 15
TC_ROWS = TC_NBLK * STEP_ROWS
TAIL_START = 999424
TAIL_N = LTM_N - TAIL_START

NWORK = 32
SC_START = TC_ROWS
WROWS = 15872
CHUNK = 992
NCHUNK = 16
GROUPS = CHUNK // 16
assert SC_START + NWORK * WROWS == TAIL_START


def _scalar(x2d):
    return x2d[0, 0]


def _v_rsqrt(a):
    ai = plsc.bitcast(a, jnp.int32)
    yi = 0x5F3759DF - lax.shift_right_logical(ai, 1)
    y = plsc.bitcast(yi, jnp.float32)
    for _ in range(3):
        y = y * (1.5 - 0.5 * a * y * y)
    return y


def _merge_scalar(run_v, run_i, cv, ci):
    v0, v1, v2 = run_v[0], run_v[1], run_v[2]
    i0, i1, i2 = run_i[0], run_i[1], run_i[2]

    def better(rv, ri):
        return (cv > rv) | ((cv == rv) & (ci < ri))

    b0, b1, b2 = better(v0, i0), better(v1, i1), better(v2, i2)
    run_v[0] = jnp.where(b0, cv, v0)
    run_i[0] = jnp.where(b0, ci, i0)
    run_v[1] = jnp.where(b0, v0, jnp.where(b1, cv, v1))
    run_i[1] = jnp.where(b0, i0, jnp.where(b1, ci, i1))
    run_v[2] = jnp.where(b1, v1, jnp.where(b2, cv, v2))
    run_i[2] = jnp.where(b1, i1, jnp.where(b2, ci, i2))


def _top3_tc(vals2d, gidx2d, alive0):
    alive = alive0
    out_v, out_i = [], []
    for _ in range(K):
        masked = jnp.where(alive, vals2d, NEG_INF)
        m2d = jnp.max(masked, keepdims=True)
        sel = alive & (masked == m2d)
        i2d = jnp.min(jnp.where(sel, gidx2d, BIG_I32), keepdims=True)
        out_v.append(_scalar(m2d))
        out_i.append(_scalar(i2d))
        alive = alive & (gidx2d != i2d)
    return out_v, out_i


def _make_sc_scan():
    mesh = plsc.VectorSubcoreMesh(core_axis_name="c", subcore_axis_name="s")

    @functools.partial(
        pl.kernel,
        mesh=mesh,
        out_type=[
            jax.ShapeDtypeStruct((NWORK, 16), jnp.float32),
            jax.ShapeDtypeStruct((NWORK, 16), jnp.int32),
        ],
        scratch_types=[
            pltpu.VMEM((CHUNK * EMB_DIM,), jnp.float32),
            pltpu.VMEM((EMB_DIM, 16), jnp.float32),
            pltpu.VMEM((16,), jnp.float32),
            pltpu.VMEM((16,), jnp.int32),
            pltpu.SMEM((4,), jnp.float32),
            pltpu.SMEM((4,), jnp.int32),
        ],
        compiler_params=pltpu.CompilerParams(needs_layout_passes=False),
    )
    def sc_scan(ltm_flat_hbm, qrot_hbm, vals_out, idx_out,
                buf, qrot_v, vstage, istage, run_v, run_i):
        wid = lax.axis_index("s") * 2 + lax.axis_index("c")
        base = SC_START + wid * WROWS
        pltpu.sync_copy(qrot_hbm, qrot_v)
        for k in range(K):
            run_v[k] = NEG_INF
            run_i[k] = 0

        lane = lax.iota(jnp.int32, 16)
        lane64 = lane * EMB_DIM

        def chunk_body(c, carry):
            cb = base + c * CHUNK
            pltpu.sync_copy(
                ltm_flat_hbm.at[pl.ds(cb * EMB_DIM, CHUNK * EMB_DIM)], buf)

            def grp_body(g, carry2):
                bvec = lane64 + g * (16 * EMB_DIM)
                zero = jnp.zeros((16,), jnp.float32)
                dacc = [zero, zero, zero, zero]
                nacc = [zero, zero, zero, zero]
                for j in range(EMB_DIM):
                    rot = (lane + j) & (EMB_DIM - 1)
                    col = plsc.load_gather(buf, [bvec + rot])
                    qj = qrot_v[j, :]
                    dacc[j % 4] = dacc[j % 4] + col * qj
                    nacc[j % 4] = nacc[j % 4] + col * col
                dot = (dacc[0] + dacc[1]) + (dacc[2] + dacc[3])
                n2 = (nacc[0] + nacc[1]) + (nacc[2] + nacc[3])
                sims = dot * _v_rsqrt(jnp.maximum(n2, 1e-30))
                m = jnp.max(sims)

                @pl.when(m > run_v[2])
                def _extract():
                    masked = sims
                    for _ in range(K):
                        mk = jnp.max(masked)
                        lk = jnp.min(jnp.where(masked == mk, lane, BIG_I32))
                        gk = cb + g * 16 + lk
                        _merge_scalar(run_v, run_i, mk, gk)
                        masked = jnp.where(lane == lk, NEG_INF, masked)

                return carry2

            lax.fori_loop(0, GROUPS, grp_body, 0)
            return carry

        lax.fori_loop(0, NCHUNK, chunk_body, 0)

        vv = jnp.where(lane == 0, run_v[0],
                       jnp.where(lane == 1, run_v[1],
                                 jnp.where(lane == 2, run_v[2], NEG_INF)))
        iv = jnp.where(lane == 0, run_i[0],
                       jnp.where(lane == 1, run_i[1],
                                 jnp.where(lane == 2, run_i[2], BIG_I32)))
        vstage[...] = vv
        istage[...] = iv
        pltpu.sync_copy(vstage, vals_out.at[wid])
        pltpu.sync_copy(istage, idx_out.at[wid])

    return sc_scan


def _tc_scan_body(qpad2_ref, qpad64_ref, *rest):
    x_refs = rest[:NSTREAM]
    tail_ref = rest[NSTREAM]
    tcv_out, tci_out = rest[NSTREAM + 1:NSTREAM + 3]
    run_v, run_i = rest[NSTREAM + 3:NSTREAM + 5]

    i = pl.program_id(0)

    @pl.when(i == 0)
    def _init():
        for k in range(K):
            run_v[k] = NEG_INF
            run_i[k] = 0

    qpad2 = qpad2_ref[...]
    sims_parts = []
    for j in range(NSTREAM):
        x = x_refs[j][...]
        d8 = jax.lax.dot_general(qpad2, x, DN_T,
                                 preferred_element_type=jnp.float32)
        n8 = jax.lax.dot_general(qpad2, x * x, DN_T,
                                 preferred_element_type=jnp.float32)
        sims_parts.append(d8[0:2, :]
                          * jax.lax.rsqrt(jnp.maximum(n8[2:4, :], 1e-30)))
    sims = jnp.concatenate(sims_parts, axis=0)
    mx = _scalar(jnp.max(sims, keepdims=True))

    @pl.when(mx > run_v[2])
    def _extract():
        rr = jax.lax.broadcasted_iota(jnp.int32, (2 * NSTREAM, XSUB), 0)
        cc = jax.lax.broadcasted_iota(jnp.int32, (2 * NSTREAM, XSUB), 1)
        gidx = (i * STEP_ROWS + (rr >> 1) * SUB_ROWS + cc * 2 + (rr & 1))
        cand_v, cand_i = _top3_tc(sims, gidx, gidx < BIG_I32)
        for k in range(K):
            _merge_scalar(run_v, run_i, cand_v[k], cand_i[k])

    @pl.when(i == TC_NBLK - 1)
    def _final():
        qpad64 = qpad64_ref[...]
        tail = tail_ref[...]
        d8 = jax.lax.dot_general(qpad64, tail, DN_T,
                                 preferred_element_type=jnp.float32)
        n8 = jax.lax.dot_general(qpad64, tail * tail, DN_T,
                                 preferred_element_type=jnp.float32)
        tsims = d8[0:1, :] * jax.lax.rsqrt(jnp.maximum(n8[1:2, :], 1e-30))
        tgidx = (jax.lax.broadcasted_iota(jnp.int32, (1, TAIL_N), 1)
                 + TAIL_START)
        tv, ti = _top3_tc(tsims, tgidx, tgidx < BIG_I32)
        for k in range(K):
            _merge_scalar(run_v, run_i, tv[k], ti[k])
        for k in range(16):
            tcv_out[0, k] = run_v[k] if k < K else NEG_INF
            tci_out[0, k] = run_i[k] if k < K else BIG_I32


def _tc_scan(qpad2, qpad64, ltm_x, ltm_tail):
    def _xmap(j):
        return lambda i: (NSTREAM * i + j, 0)

    return pl.pallas_call(
        _tc_scan_body,
        grid=(TC_NBLK,),
        in_specs=[
            pl.BlockSpec((8, 2 * EMB_DIM), lambda i: (0, 0)),
            pl.BlockSpec((8, EMB_DIM), lambda i: (0, 0)),
        ] + [
            pl.BlockSpec((XSUB, 2 * EMB_DIM), _xmap(j)) for j in range(NSTREAM)
        ] + [
            pl.BlockSpec((TAIL_N, EMB_DIM), lambda i: (0, 0)),
        ],
        out_specs=(
            pl.BlockSpec(memory_space=pltpu.SMEM),
            pl.BlockSpec(memory_space=pltpu.SMEM),
        ),
        out_shape=(
            jax.ShapeDtypeStruct((1, 16), jnp.float32),
            jax.ShapeDtypeStruct((1, 16), jnp.int32),
        ),
        scratch_shapes=[
            pltpu.SMEM((4,), jnp.float32),
            pltpu.SMEM((4,), jnp.int32),
        ],
        compiler_params=pltpu.CompilerParams(
            dimension_semantics=("arbitrary",)),
    )(qpad2, qpad64, *([ltm_x] * NSTREAM), ltm_tail)


def _finish_body(q_ref, qpad64_ref, qrel_ref, node_ref, stm_e_ref, stm_r_ref,
                 cv_ref, ci_ref, ltm_e_hbm, ltm_p_hbm,
                 emb_out, pos_out, sco_out, src_out, sem):
    q = q_ref[...]
    qpad64 = qpad64_ref[...]
    qn2 = _scalar(jnp.sum(q * q, keepdims=True))
    qinv = 1.0 / (jnp.sqrt(qn2) + EPS)

    cvals = cv_ref[...]
    cidx = ci_ref[...]
    lv, li = _top3_tc(cvals, cidx, cidx < BIG_I32)

    qrel = qrel_ref[...]
    stm_r = stm_r_ref[...]
    diff = stm_r - qrel
    d2 = jnp.sum(diff * diff, axis=1)
    within = (d2 <= RADIUS2).reshape(1, STM_CAP)
    stm_e = stm_e_ref[...]
    sd8 = jax.lax.dot_general(qpad64, stm_e, DN_T,
                              preferred_element_type=jnp.float32)
    sn8 = jax.lax.dot_general(qpad64, stm_e * stm_e, DN_T,
                              preferred_element_type=jnp.float32)
    ssim = (sd8[0:1, :] / (jnp.sqrt(sn8[1:2, :]) + EPS)) * qinv
    ssim2 = jnp.where(within, ssim, NEG_INF)
    scol = jax.lax.broadcasted_iota(jnp.int32, (1, STM_CAP), 1)
    sv, si = _top3_tc(ssim2, scol, scol < BIG_I32)

    stm_hit = sv[0] >= SIM_THRESHOLD
    src_out[0, 0] = jnp.where(stm_hit, 1.0, 0.0).astype(jnp.float32)
    for k in range(K):
        sco_out[0, k] = jnp.where(stm_hit, sv[k], lv[k] * qinv)

    @pl.when(stm_hit)
    def _stm_write():
        for k in range(K):
            cp = pltpu.make_async_copy(
                stm_e_ref.at[pl.ds(si[k], 1)], emb_out.at[pl.ds(k, 1)], sem)
            cp.start()
            cp.wait()
            cp = pltpu.make_async_copy(
                stm_r_ref.at[pl.ds(si[k], 1)], pos_out.at[pl.ds(k, 1)], sem)
            cp.start()
            cp.wait()
        pos_out[...] = pos_out[...] + node_ref[...]

    @pl.when(jnp.logical_not(stm_hit))
    def _ltm_write():
        for k in range(K):
            cp = pltpu.make_async_copy(
                ltm_e_hbm.at[pl.ds(li[k], 1)], emb_out.at[pl.ds(k, 1)], sem)
            cp.start()
            cp.wait()
            cp = pltpu.make_async_copy(
                ltm_p_hbm.at[pl.ds(li[k], 1)], pos_out.at[pl.ds(k, 1)], sem)
            cp.start()
            cp.wait()


def kernel(current_observation_embedding, current_absolute_position,
           current_semantic_node_position, stm_embeddings, stm_rel_positions,
           ltm_embeddings, ltm_positions):
    q = current_observation_embedding
    q2 = q.reshape(1, EMB_DIM)
    qpad2 = jnp.zeros((8, 2 * EMB_DIM), jnp.float32)
    qpad2 = qpad2.at[0, :EMB_DIM].set(q)
    qpad2 = qpad2.at[1, EMB_DIM:].set(q)
    qpad2 = qpad2.at[2, :EMB_DIM].set(1.0)
    qpad2 = qpad2.at[3, EMB_DIM:].set(1.0)
    qpad64 = jnp.zeros((8, EMB_DIM), jnp.float32)
    qpad64 = qpad64.at[0, :].set(q)
    qpad64 = qpad64.at[1, :].set(1.0)
    rot = (jnp.arange(EMB_DIM)[:, None] + jnp.arange(16)[None, :]) % EMB_DIM
    qrot = q[rot]
    qrel = (current_absolute_position - current_semantic_node_position).reshape(1, 3)
    node = current_semantic_node_position.reshape(1, 3)
    ltm_x = ltm_embeddings.reshape(LTM_N // 2, 2 * EMB_DIM)
    ltm_tail = ltm_embeddings[TAIL_START:, :]

    sc_scan = _make_sc_scan()
    scv, sci = sc_scan(ltm_embeddings.reshape(LTM_N * EMB_DIM), qrot)
    tcv, tci = _tc_scan(qpad2, qpad64, ltm_x, ltm_tail)

    cvals = jnp.concatenate([tcv, scv.reshape(1, NWORK * 16)], axis=1)
    cidx = jnp.concatenate([tci, sci.reshape(1, NWORK * 16)], axis=1)
    ncand = 16 + NWORK * 16

    out_shape = (
        jax.ShapeDtypeStruct((K, EMB_DIM), jnp.float32),
        jax.ShapeDtypeStruct((K, 3), jnp.float32),
        jax.ShapeDtypeStruct((1, K), jnp.float32),
        jax.ShapeDtypeStruct((1, 1), jnp.float32),
    )
    emb, pos, sco, src = pl.pallas_call(
        _finish_body,
        grid=(1,),
        in_specs=[
            pl.BlockSpec((1, EMB_DIM), lambda i: (0, 0)),
            pl.BlockSpec((8, EMB_DIM), lambda i: (0, 0)),
            pl.BlockSpec((1, 3), lambda i: (0, 0)),
            pl.BlockSpec((1, 3), lambda i: (0, 0)),
            pl.BlockSpec((STM_CAP, EMB_DIM), lambda i: (0, 0)),
            pl.BlockSpec((STM_CAP, 3), lambda i: (0, 0)),
            pl.BlockSpec((1, ncand), lambda i: (0, 0)),
            pl.BlockSpec((1, ncand), lambda i: (0, 0)),
            pl.BlockSpec(memory_space=pl.ANY),
            pl.BlockSpec(memory_space=pl.ANY),
        ],
        out_specs=(
            pl.BlockSpec((K, EMB_DIM), lambda i: (0, 0)),
            pl.BlockSpec((K, 3), lambda i: (0, 0)),
            pl.BlockSpec(memory_space=pltpu.SMEM),
            pl.BlockSpec(memory_space=pltpu.SMEM),
        ),
        out_shape=out_shape,
        scratch_shapes=[pltpu.SemaphoreType.DMA],
    )(q2, qpad64, qrel, node, stm_embeddings, stm_rel_positions,
      cvals, cidx, ltm_embeddings, ltm_positions)
    return emb, pos, sco.reshape(K), src.reshape(())


# speedup vs baseline: 1.0003x; 1.0003x over previous
import functools

import jax
import jax.numpy as jnp
from jax import lax
from jax.experimental import pallas as pl
from jax.experimental.pallas import tpu as pltpu
from jax.experimental.pallas import tpu_sc as plsc

EMB_DIM = 64
LTM_N = 1000000
STM_CAP = 128
K = 3
RADIUS2 = 9.0
SIM_THRESHOLD = 0.7
EPS = 1e-8
NEG_INF = float("-inf")
BIG_I32 = 1 << 30
DN_T = (((1,), (1,)), ((), ()))

NSTREAM = 4
STEP_ROWS = 32768
SUB_ROWS = STEP_ROWS // NSTREAM
XSUB = SUB_ROWS // 2
TC_NBLK = 15
TC_ROWS = TC_NBLK * STEP_ROWS
TAIL_START = 999424
TAIL_N = LTM_N - TAIL_START

NWORK = 32
SC_START = TC_ROWS
WROWS = 15872
CHUNK = 496
NCHUNK = 32
GROUPS = CHUNK // 16
assert SC_START + NWORK * WROWS == TAIL_START


def _scalar(x2d):
    return x2d[0, 0]


def _v_rsqrt(a):
    ai = plsc.bitcast(a, jnp.int32)
    yi = 0x5F3759DF - lax.shift_right_logical(ai, 1)
    y = plsc.bitcast(yi, jnp.float32)
    for _ in range(3):
        y = y * (1.5 - 0.5 * a * y * y)
    return y


def _merge_scalar(run_v, run_i, cv, ci):
    v0, v1, v2 = run_v[0], run_v[1], run_v[2]
    i0, i1, i2 = run_i[0], run_i[1], run_i[2]

    def better(rv, ri):
        return (cv > rv) | ((cv == rv) & (ci < ri))

    b0, b1, b2 = better(v0, i0), better(v1, i1), better(v2, i2)
    run_v[0] = jnp.where(b0, cv, v0)
    run_i[0] = jnp.where(b0, ci, i0)
    run_v[1] = jnp.where(b0, v0, jnp.where(b1, cv, v1))
    run_i[1] = jnp.where(b0, i0, jnp.where(b1, ci, i1))
    run_v[2] = jnp.where(b1, v1, jnp.where(b2, cv, v2))
    run_i[2] = jnp.where(b1, i1, jnp.where(b2, ci, i2))


def _top3_tc(vals2d, gidx2d, alive0):
    alive = alive0
    out_v, out_i = [], []
    for _ in range(K):
        masked = jnp.where(alive, vals2d, NEG_INF)
        m2d = jnp.max(masked, keepdims=True)
        sel = alive & (masked == m2d)
        i2d = jnp.min(jnp.where(sel, gidx2d, BIG_I32), keepdims=True)
        out_v.append(_scalar(m2d))
        out_i.append(_scalar(i2d))
        alive = alive & (gidx2d != i2d)
    return out_v, out_i


def _make_sc_scan():
    mesh = plsc.VectorSubcoreMesh(core_axis_name="c", subcore_axis_name="s")

    @functools.partial(
        pl.kernel,
        mesh=mesh,
        out_type=[
            jax.ShapeDtypeStruct((NWORK, 16), jnp.float32),
            jax.ShapeDtypeStruct((NWORK, 16), jnp.int32),
        ],
        scratch_types=[
            pltpu.VMEM((CHUNK * EMB_DIM,), jnp.float32),
            pltpu.VMEM((CHUNK * EMB_DIM,), jnp.float32),
            pltpu.VMEM((EMB_DIM, 16), jnp.float32),
            pltpu.VMEM((16,), jnp.float32),
            pltpu.VMEM((16,), jnp.int32),
            pltpu.SMEM((4,), jnp.float32),
            pltpu.SMEM((4,), jnp.int32),
            pltpu.SemaphoreType.DMA,
            pltpu.SemaphoreType.DMA,
        ],
        compiler_params=pltpu.CompilerParams(needs_layout_passes=False),
    )
    def sc_scan(ltm_flat_hbm, qrot_hbm, vals_out, idx_out,
                buf_a, buf_b, qrot_v, vstage, istage, run_v, run_i,
                sem_a, sem_b):
        wid = lax.axis_index("s") * 2 + lax.axis_index("c")
        base = SC_START + wid * WROWS
        pltpu.sync_copy(qrot_hbm, qrot_v)
        for k in range(K):
            run_v[k] = NEG_INF
            run_i[k] = 0

        lane = lax.iota(jnp.int32, 16)
        lane64 = lane * EMB_DIM

        def _src(c):
            return ltm_flat_hbm.at[pl.ds((base + c * CHUNK) * EMB_DIM,
                                         CHUNK * EMB_DIM)]

        def _process(buf, cb):
            def grp_body(g, carry2):
                bvec = lane64 + g * (16 * EMB_DIM)
                zero = jnp.zeros((16,), jnp.float32)
                dacc = [zero, zero, zero, zero]
                nacc = [zero, zero, zero, zero]
                for j in range(EMB_DIM):
                    rot = (lane + j) & (EMB_DIM - 1)
                    col = plsc.load_gather(buf, [bvec + rot])
                    qj = qrot_v[j, :]
                    dacc[j % 4] = dacc[j % 4] + col * qj
                    nacc[j % 4] = nacc[j % 4] + col * col
                dot = (dacc[0] + dacc[1]) + (dacc[2] + dacc[3])
                n2 = (nacc[0] + nacc[1]) + (nacc[2] + nacc[3])
                sims = dot * _v_rsqrt(jnp.maximum(n2, 1e-30))
                m = jnp.max(sims)

                @pl.when(m > run_v[2])
                def _extract():
                    masked = sims
                    for _ in range(K):
                        mk = jnp.max(masked)
                        lk = jnp.min(jnp.where(masked == mk, lane, BIG_I32))
                        gk = cb + g * 16 + lk
                        _merge_scalar(run_v, run_i, mk, gk)
                        masked = jnp.where(lane == lk, NEG_INF, masked)

                return carry2

            lax.fori_loop(0, GROUPS, grp_body, 0)

        pltpu.async_copy(_src(0), buf_a, sem_a)

        def pair_body(it, carry):
            c = it * 2
            pltpu.make_async_copy(_src(c), buf_a, sem_a).wait()
            pltpu.async_copy(_src(c + 1), buf_b, sem_b)
            _process(buf_a, base + c * CHUNK)
            pltpu.make_async_copy(_src(c + 1), buf_b, sem_b).wait()

            @pl.when(c + 2 < NCHUNK)
            def _start_next():
                pltpu.async_copy(_src(c + 2), buf_a, sem_a)

            _process(buf_b, base + (c + 1) * CHUNK)
            return carry

        lax.fori_loop(0, NCHUNK // 2, pair_body, 0)

        vv = jnp.where(lane == 0, run_v[0],
                       jnp.where(lane == 1, run_v[1],
                                 jnp.where(lane == 2, run_v[2], NEG_INF)))
        iv = jnp.where(lane == 0, run_i[0],
                       jnp.where(lane == 1, run_i[1],
                                 jnp.where(lane == 2, run_i[2], BIG_I32)))
        vstage[...] = vv
        istage[...] = iv
        pltpu.sync_copy(vstage, vals_out.at[wid])
        pltpu.sync_copy(istage, idx_out.at[wid])

    return sc_scan


def _tc_scan_body(qpad2_ref, qpad64_ref, *rest):
    x_refs = rest[:NSTREAM]
    tail_ref = rest[NSTREAM]
    tcv_out, tci_out = rest[NSTREAM + 1:NSTREAM + 3]
    run_v, run_i = rest[NSTREAM + 3:NSTREAM + 5]

    i = pl.program_id(0)

    @pl.when(i == 0)
    def _init():
        for k in range(K):
            run_v[k] = NEG_INF
            run_i[k] = 0

    qpad2 = qpad2_ref[...]
    sims_parts = []
    for j in range(NSTREAM):
        x = x_refs[j][...]
        d8 = jax.lax.dot_general(qpad2, x, DN_T,
                                 preferred_element_type=jnp.float32)
        n8 = jax.lax.dot_general(qpad2, x * x, DN_T,
                                 preferred_element_type=jnp.float32)
        sims_parts.append(d8[0:2, :]
                          * jax.lax.rsqrt(jnp.maximum(n8[2:4, :], 1e-30)))
    sims = jnp.concatenate(sims_parts, axis=0)
    mx = _scalar(jnp.max(sims, keepdims=True))

    @pl.when(mx > run_v[2])
    def _extract():
        rr = jax.lax.broadcasted_iota(jnp.int32, (2 * NSTREAM, XSUB), 0)
        cc = jax.lax.broadcasted_iota(jnp.int32, (2 * NSTREAM, XSUB), 1)
        gidx = (i * STEP_ROWS + (rr >> 1) * SUB_ROWS + cc * 2 + (rr & 1))
        cand_v, cand_i = _top3_tc(sims, gidx, gidx < BIG_I32)
        for k in range(K):
            _merge_scalar(run_v, run_i, cand_v[k], cand_i[k])

    @pl.when(i == TC_NBLK - 1)
    def _final():
        qpad64 = qpad64_ref[...]
        tail = tail_ref[...]
        d8 = jax.lax.dot_general(qpad64, tail, DN_T,
                                 preferred_element_type=jnp.float32)
        n8 = jax.lax.dot_general(qpad64, tail * tail, DN_T,
                                 preferred_element_type=jnp.float32)
        tsims = d8[0:1, :] * jax.lax.rsqrt(jnp.maximum(n8[1:2, :], 1e-30))
        tgidx = (jax.lax.broadcasted_iota(jnp.int32, (1, TAIL_N), 1)
                 + TAIL_START)
        tv, ti = _top3_tc(tsims, tgidx, tgidx < BIG_I32)
        for k in range(K):
            _merge_scalar(run_v, run_i, tv[k], ti[k])
        for k in range(16):
            tcv_out[0, k] = run_v[k] if k < K else NEG_INF
            tci_out[0, k] = run_i[k] if k < K else BIG_I32


def _tc_scan(qpad2, qpad64, ltm_x, ltm_tail):
    def _xmap(j):
        return lambda i: (NSTREAM * i + j, 0)

    return pl.pallas_call(
        _tc_scan_body,
        grid=(TC_NBLK,),
        in_specs=[
            pl.BlockSpec((8, 2 * EMB_DIM), lambda i: (0, 0)),
            pl.BlockSpec((8, EMB_DIM), lambda i: (0, 0)),
        ] + [
            pl.BlockSpec((XSUB, 2 * EMB_DIM), _xmap(j)) for j in range(NSTREAM)
        ] + [
            pl.BlockSpec((TAIL_N, EMB_DIM), lambda i: (0, 0)),
        ],
        out_specs=(
            pl.BlockSpec(memory_space=pltpu.SMEM),
            pl.BlockSpec(memory_space=pltpu.SMEM),
        ),
        out_shape=(
            jax.ShapeDtypeStruct((1, 16), jnp.float32),
            jax.ShapeDtypeStruct((1, 16), jnp.int32),
        ),
        scratch_shapes=[
            pltpu.SMEM((4,), jnp.float32),
            pltpu.SMEM((4,), jnp.int32),
        ],
        compiler_params=pltpu.CompilerParams(
            dimension_semantics=("arbitrary",)),
    )(qpad2, qpad64, *([ltm_x] * NSTREAM), ltm_tail)


def _finish_body(q_ref, qpad64_ref, qrel_ref, node_ref, stm_e_ref, stm_r_ref,
                 cv_ref, ci_ref, ltm_e_hbm, ltm_p_hbm,
                 emb_out, pos_out, sco_out, src_out, sem):
    q = q_ref[...]
    qpad64 = qpad64_ref[...]
    qn2 = _scalar(jnp.sum(q * q, keepdims=True))
    qinv = 1.0 / (jnp.sqrt(qn2) + EPS)

    cvals = cv_ref[...]
    cidx = ci_ref[...]
    lv, li = _top3_tc(cvals, cidx, cidx < BIG_I32)

    qrel = qrel_ref[...]
    stm_r = stm_r_ref[...]
    diff = stm_r - qrel
    d2 = jnp.sum(diff * diff, axis=1)
    within = (d2 <= RADIUS2).reshape(1, STM_CAP)
    stm_e = stm_e_ref[...]
    sd8 = jax.lax.dot_general(qpad64, stm_e, DN_T,
                              preferred_element_type=jnp.float32)
    sn8 = jax.lax.dot_general(qpad64, stm_e * stm_e, DN_T,
                              preferred_element_type=jnp.float32)
    ssim = (sd8[0:1, :] / (jnp.sqrt(sn8[1:2, :]) + EPS)) * qinv
    ssim2 = jnp.where(within, ssim, NEG_INF)
    scol = jax.lax.broadcasted_iota(jnp.int32, (1, STM_CAP), 1)
    sv, si = _top3_tc(ssim2, scol, scol < BIG_I32)

    stm_hit = sv[0] >= SIM_THRESHOLD
    src_out[0, 0] = jnp.where(stm_hit, 1.0, 0.0).astype(jnp.float32)
    for k in range(K):
        sco_out[0, k] = jnp.where(stm_hit, sv[k], lv[k] * qinv)

    @pl.when(stm_hit)
    def _stm_write():
        for k in range(K):
            cp = pltpu.make_async_copy(
                stm_e_ref.at[pl.ds(si[k], 1)], emb_out.at[pl.ds(k, 1)], sem)
            cp.start()
            cp.wait()
            cp = pltpu.make_async_copy(
                stm_r_ref.at[pl.ds(si[k], 1)], pos_out.at[pl.ds(k, 1)], sem)
            cp.start()
            cp.wait()
        pos_out[...] = pos_out[...] + node_ref[...]

    @pl.when(jnp.logical_not(stm_hit))
    def _ltm_write():
        for k in range(K):
            cp = pltpu.make_async_copy(
                ltm_e_hbm.at[pl.ds(li[k], 1)], emb_out.at[pl.ds(k, 1)], sem)
            cp.start()
            cp.wait()
            cp = pltpu.make_async_copy(
                ltm_p_hbm.at[pl.ds(li[k], 1)], pos_out.at[pl.ds(k, 1)], sem)
            cp.start()
            cp.wait()


def kernel(current_observation_embedding, current_absolute_position,
           current_semantic_node_position, stm_embeddings, stm_rel_positions,
           ltm_embeddings, ltm_positions):
    q = current_observation_embedding
    q2 = q.reshape(1, EMB_DIM)
    qpad2 = jnp.zeros((8, 2 * EMB_DIM), jnp.float32)
    qpad2 = qpad2.at[0, :EMB_DIM].set(q)
    qpad2 = qpad2.at[1, EMB_DIM:].set(q)
    qpad2 = qpad2.at[2, :EMB_DIM].set(1.0)
    qpad2 = qpad2.at[3, EMB_DIM:].set(1.0)
    qpad64 = jnp.zeros((8, EMB_DIM), jnp.float32)
    qpad64 = qpad64.at[0, :].set(q)
    qpad64 = qpad64.at[1, :].set(1.0)
    rot = (jnp.arange(EMB_DIM)[:, None] + jnp.arange(16)[None, :]) % EMB_DIM
    qrot = q[rot]
    qrel = (current_absolute_position - current_semantic_node_position).reshape(1, 3)
    node = current_semantic_node_position.reshape(1, 3)
    ltm_x = ltm_embeddings.reshape(LTM_N // 2, 2 * EMB_DIM)
    ltm_tail = ltm_embeddings[TAIL_START:, :]

    sc_scan = _make_sc_scan()
    scv, sci = sc_scan(ltm_embeddings.reshape(LTM_N * EMB_DIM), qrot)
    tcv, tci = _tc_scan(qpad2, qpad64, ltm_x, ltm_tail)

    cvals = jnp.concatenate([tcv, scv.reshape(1, NWORK * 16)], axis=1)
    cidx = jnp.concatenate([tci, sci.reshape(1, NWORK * 16)], axis=1)
    ncand = 16 + NWORK * 16

    out_shape = (
        jax.ShapeDtypeStruct((K, EMB_DIM), jnp.float32),
        jax.ShapeDtypeStruct((K, 3), jnp.float32),
        jax.ShapeDtypeStruct((1, K), jnp.float32),
        jax.ShapeDtypeStruct((1, 1), jnp.float32),
    )
    emb, pos, sco, src = pl.pallas_call(
        _finish_body,
        grid=(1,),
        in_specs=[
            pl.BlockSpec((1, EMB_DIM), lambda i: (0, 0)),
            pl.BlockSpec((8, EMB_DIM), lambda i: (0, 0)),
            pl.BlockSpec((1, 3), lambda i: (0, 0)),
            pl.BlockSpec((1, 3), lambda i: (0, 0)),
            pl.BlockSpec((STM_CAP, EMB_DIM), lambda i: (0, 0)),
            pl.BlockSpec((STM_CAP, 3), lambda i: (0, 0)),
            pl.BlockSpec((1, ncand), lambda i: (0, 0)),
            pl.BlockSpec((1, ncand), lambda i: (0, 0)),
            pl.BlockSpec(memory_space=pl.ANY),
            pl.BlockSpec(memory_space=pl.ANY),
        ],
        out_specs=(
            pl.BlockSpec((K, EMB_DIM), lambda i: (0, 0)),
            pl.BlockSpec((K, 3), lambda i: (0, 0)),
            pl.BlockSpec(memory_space=pltpu.SMEM),
            pl.BlockSpec(memory_space=pltpu.SMEM),
        ),
        out_shape=out_shape,
        scratch_shapes=[pltpu.SemaphoreType.DMA],
    )(q2, qpad64, qrel, node, stm_embeddings, stm_rel_positions,
      cvals, cidx, ltm_embeddings, ltm_positions)
    return emb, pos, sco.reshape(K), src.reshape(())


# speedup vs baseline: 1.3430x; 1.3426x over previous
import jax
import jax.numpy as jnp
from jax.experimental import pallas as pl
from jax.experimental.pallas import tpu as pltpu

EMB_DIM = 64
LTM_N = 1000000
STM_CAP = 128
K = 3
RADIUS2 = 9.0
SIM_THRESHOLD = 0.7
EPS = 1e-8
NSTREAM = 4
STEP_ROWS = 32768
SUB_ROWS = STEP_ROWS // NSTREAM
XSUB = SUB_ROWS // 2
NBLK = 30
TAIL_START = NBLK * STEP_ROWS
TAIL_N = LTM_N - TAIL_START
NEG_INF = float("-inf")
BIG_I32 = 1 << 30
DN_T = (((1,), (1,)), ((), ()))


def _scalar(x2d):
    return x2d[0, 0]


def _top3(vals2d, gidx2d, alive0):
    alive = alive0
    out_v, out_i = [], []
    for _ in range(K):
        masked = jnp.where(alive, vals2d, NEG_INF)
        m2d = jnp.max(masked, keepdims=True)
        sel = alive & (masked == m2d)
        i2d = jnp.min(jnp.where(sel, gidx2d, BIG_I32), keepdims=True)
        out_v.append(_scalar(m2d))
        out_i.append(_scalar(i2d))
        alive = alive & (gidx2d != i2d)
    return out_v, out_i


def _merge_candidate(run_v, run_i, cv, ci):
    v0, v1, v2 = run_v[0], run_v[1], run_v[2]
    i0, i1, i2 = run_i[0], run_i[1], run_i[2]

    def better(rv, ri):
        return (cv > rv) | ((cv == rv) & (ci < ri))

    b0, b1, b2 = better(v0, i0), better(v1, i1), better(v2, i2)
    run_v[0] = jnp.where(b0, cv, v0)
    run_i[0] = jnp.where(b0, ci, i0)
    run_v[1] = jnp.where(b0, v0, jnp.where(b1, cv, v1))
    run_i[1] = jnp.where(b0, i0, jnp.where(b1, ci, i1))
    run_v[2] = jnp.where(b1, v1, jnp.where(b2, cv, v2))
    run_i[2] = jnp.where(b1, i1, jnp.where(b2, ci, i2))


def _sims_transposed(qpad, mat):
    d8 = jax.lax.dot_general(qpad, mat, DN_T,
                             preferred_element_type=jnp.float32)
    n8 = jax.lax.dot_general(qpad, mat * mat, DN_T,
                             preferred_element_type=jnp.float32)
    return d8[0:1, :], n8[1:2, :]


def _body(*refs):
    (q_ref, qpad2_ref, qpad64_ref, qrel_ref, node_ref, stm_e_ref,
     stm_r_ref) = refs[:7]
    x_refs = refs[7:7 + NSTREAM]
    tail_ref, ltm_e_hbm, ltm_p_hbm = refs[7 + NSTREAM:10 + NSTREAM]
    emb_out, pos_out, sco_out, src_out = refs[10 + NSTREAM:14 + NSTREAM]
    run_v, run_i, sem = refs[14 + NSTREAM:]

    i = pl.program_id(0)

    @pl.when(i == 0)
    def _init():
        for k in range(K):
            run_v[k] = NEG_INF
            run_i[k] = 0

    qpad2 = qpad2_ref[...]
    sims_parts = []
    for j in range(NSTREAM):
        x = x_refs[j][...]
        d8 = jax.lax.dot_general(qpad2, x, DN_T,
                                 preferred_element_type=jnp.float32)
        n8 = jax.lax.dot_general(qpad2, x * x, DN_T,
                                 preferred_element_type=jnp.float32)
        sims_parts.append(d8[0:2, :]
                          * jax.lax.rsqrt(jnp.maximum(n8[2:4, :], 1e-30)))
    sims = jnp.concatenate(sims_parts, axis=0)
    mx = _scalar(jnp.max(sims, keepdims=True))

    @pl.when(mx > run_v[2])
    def _extract():
        rr = jax.lax.broadcasted_iota(jnp.int32, (2 * NSTREAM, XSUB), 0)
        cc = jax.lax.broadcasted_iota(jnp.int32, (2 * NSTREAM, XSUB), 1)
        gidx = (i * STEP_ROWS + (rr >> 1) * SUB_ROWS + cc * 2 + (rr & 1))
        cand_v, cand_i = _top3(sims, gidx, gidx < BIG_I32)
        for k in range(K):
            _merge_candidate(run_v, run_i, cand_v[k], cand_i[k])

    @pl.when(i == NBLK - 1)
    def _final():
        q = q_ref[...]
        qpad64 = qpad64_ref[...]
        qn2 = _scalar(jnp.sum(q * q, keepdims=True))
        qinv = 1.0 / (jnp.sqrt(qn2) + EPS)

        tail = tail_ref[...]
        tdots, tn2 = _sims_transposed(qpad64, tail)
        tsims = tdots * jax.lax.rsqrt(jnp.maximum(tn2, 1e-30))
        tgidx = (jax.lax.broadcasted_iota(jnp.int32, (1, TAIL_N), 1)
                 + TAIL_START)
        tv, ti = _top3(tsims, tgidx, tgidx < BIG_I32)
        for k in range(K):
            _merge_candidate(run_v, run_i, tv[k], ti[k])

        qrel = qrel_ref[...]
        stm_r = stm_r_ref[...]
        diff = stm_r - qrel
        d2 = jnp.sum(diff * diff, axis=1)
        within = (d2 <= RADIUS2).reshape(1, STM_CAP)
        stm_e = stm_e_ref[...]
        sdots, sn2 = _sims_transposed(qpad64, stm_e)
        ssim = (sdots / (jnp.sqrt(sn2) + EPS)) * qinv
        ssim2 = jnp.where(within, ssim, NEG_INF)
        scol = jax.lax.broadcasted_iota(jnp.int32, (1, STM_CAP), 1)
        sv, si = _top3(ssim2, scol, scol < BIG_I32)

        stm_hit = sv[0] >= SIM_THRESHOLD
        src_out[0, 0] = jnp.where(stm_hit, 1.0, 0.0).astype(jnp.float32)
        for k in range(K):
            sco_out[0, k] = jnp.where(stm_hit, sv[k], run_v[k] * qinv)

        @pl.when(stm_hit)
        def _stm_write():
            for k in range(K):
                cp = pltpu.make_async_copy(
                    stm_e_ref.at[pl.ds(si[k], 1)], emb_out.at[pl.ds(k, 1)], sem)
                cp.start()
                cp.wait()
                cp = pltpu.make_async_copy(
                    stm_r_ref.at[pl.ds(si[k], 1)], pos_out.at[pl.ds(k, 1)], sem)
                cp.start()
                cp.wait()
            pos_out[...] = pos_out[...] + node_ref[...]

        @pl.when(jnp.logical_not(stm_hit))
        def _ltm_write():
            for k in range(K):
                cp = pltpu.make_async_copy(
                    ltm_e_hbm.at[pl.ds(run_i[k], 1)], emb_out.at[pl.ds(k, 1)], sem)
                cp.start()
                cp.wait()
                cp = pltpu.make_async_copy(
                    ltm_p_hbm.at[pl.ds(run_i[k], 1)], pos_out.at[pl.ds(k, 1)], sem)
                cp.start()
                cp.wait()


def kernel(current_observation_embedding, current_absolute_position,
           current_semantic_node_position, stm_embeddings, stm_rel_positions,
           ltm_embeddings, ltm_positions):
    q = current_observation_embedding
    q2 = q.reshape(1, EMB_DIM)
    qpad2 = jnp.zeros((8, 2 * EMB_DIM), jnp.float32)
    qpad2 = qpad2.at[0, :EMB_DIM].set(q)
    qpad2 = qpad2.at[1, EMB_DIM:].set(q)
    qpad2 = qpad2.at[2, :EMB_DIM].set(1.0)
    qpad2 = qpad2.at[3, EMB_DIM:].set(1.0)
    qpad64 = jnp.zeros((8, EMB_DIM), jnp.float32)
    qpad64 = qpad64.at[0, :].set(q)
    qpad64 = qpad64.at[1, :].set(1.0)
    qrel = (current_absolute_position - current_semantic_node_position).reshape(1, 3)
    node = current_semantic_node_position.reshape(1, 3)
    ltm_x = ltm_embeddings.reshape(LTM_N // 2, 2 * EMB_DIM)
    ltm_tail = ltm_embeddings[TAIL_START:, :]

    def _xmap(j):
        return lambda i: (NSTREAM * i + j, 0)

    out_shape = (
        jax.ShapeDtypeStruct((K, EMB_DIM), jnp.float32),
        jax.ShapeDtypeStruct((K, 3), jnp.float32),
        jax.ShapeDtypeStruct((1, K), jnp.float32),
        jax.ShapeDtypeStruct((1, 1), jnp.float32),
    )
    emb, pos, sco, src = pl.pallas_call(
        _body,
        grid=(NBLK,),
        in_specs=[
            pl.BlockSpec((1, EMB_DIM), lambda i: (0, 0)),
            pl.BlockSpec((8, 2 * EMB_DIM), lambda i: (0, 0)),
            pl.BlockSpec((8, EMB_DIM), lambda i: (0, 0)),
            pl.BlockSpec((1, 3), lambda i: (0, 0)),
            pl.BlockSpec((1, 3), lambda i: (0, 0)),
            pl.BlockSpec((STM_CAP, EMB_DIM), lambda i: (0, 0)),
            pl.BlockSpec((STM_CAP, 3), lambda i: (0, 0)),
        ] + [
            pl.BlockSpec((XSUB, 2 * EMB_DIM), _xmap(j)) for j in range(NSTREAM)
        ] + [
            pl.BlockSpec((TAIL_N, EMB_DIM), lambda i: (0, 0)),
            pl.BlockSpec(memory_space=pl.ANY),
            pl.BlockSpec(memory_space=pl.ANY),
        ],
        out_specs=(
            pl.BlockSpec((K, EMB_DIM), lambda i: (0, 0)),
            pl.BlockSpec((K, 3), lambda i: (0, 0)),
            pl.BlockSpec(memory_space=pltpu.SMEM),
            pl.BlockSpec(memory_space=pltpu.SMEM),
        ),
        out_shape=out_shape,
        scratch_shapes=[
            pltpu.SMEM((4,), jnp.float32),
            pltpu.SMEM((4,), jnp.int32),
            pltpu.SemaphoreType.DMA,
        ],
        compiler_params=pltpu.CompilerParams(
            dimension_semantics=("arbitrary",)),
    )(q2, qpad2, qpad64, qrel, node, stm_embeddings, stm_rel_positions,
      *([ltm_x] * NSTREAM), ltm_tail, ltm_embeddings, ltm_positions)
    return emb, pos, sco.reshape(K), src.reshape(())
